# baseline (speedup 1.0000x reference)
import jax
import jax.numpy as jnp
from jax import lax
from jax.experimental import pallas as pl
from jax.experimental.pallas import tpu as pltpu


def _fused_kernel(a_ref, s_ref, w_ref, b_ref, o_ref):
    x = jnp.concatenate([a_ref[...], s_ref[...]], axis=1).astype(jnp.bfloat16)
    wv = w_ref[...].astype(jnp.bfloat16)
    acc = lax.dot_general(
        x, wv,
        dimension_numbers=(((1,), (0,)), ((), ())),
        preferred_element_type=jnp.float32)
    o_ref[...] = acc + b_ref[...]


def kernel(latent_rep, output_batch, latent_lib, w, b):
    n, h = latent_rep.shape
    k_pad, d_out = w.shape
    s_width = k_pad - h

    s = jnp.concatenate([output_batch, latent_lib], axis=1)
    pad = s_width - s.shape[1]
    if pad:
        s = jnp.pad(s, ((0, 0), (0, pad)))
    b2 = b.reshape(1, d_out)

    tm = min(1024, n)
    tn = min(1024, d_out)
    gi, gj = pl.cdiv(n, tm), pl.cdiv(d_out, tn)

    return pl.pallas_call(
        _fused_kernel,
        out_shape=jax.ShapeDtypeStruct((n, d_out), latent_rep.dtype),
        grid=(gi, gj),
        in_specs=[
            pl.BlockSpec((tm, h), lambda i, j: (i, 0)),
            pl.BlockSpec((tm, s_width), lambda i, j: (i, 0)),
            pl.BlockSpec((k_pad, tn), lambda i, j: (0, j)),
            pl.BlockSpec((1, tn), lambda i, j: (0, j)),
        ],
        out_specs=pl.BlockSpec((tm, tn), lambda i, j: (i, j)),
        compiler_params=pltpu.CompilerParams(
            dimension_semantics=("parallel", "arbitrary"),
            vmem_limit_bytes=64 << 20,
        ),
    )(latent_rep, s, w, b2)


# speedup vs baseline: 1.2858x; 1.2858x over previous
import jax
import jax.numpy as jnp
from jax import lax
from jax.experimental import pallas as pl
from jax.experimental.pallas import tpu as pltpu


def _fused_kernel(a_ref, s_ref, w_ref, b_ref, o_ref):
    x = jnp.concatenate([a_ref[...], s_ref[...]], axis=1).astype(jnp.bfloat16)
    wv = w_ref[...].astype(jnp.bfloat16)
    acc = lax.dot_general(
        x, wv,
        dimension_numbers=(((1,), (0,)), ((), ())),
        preferred_element_type=jnp.float32)
    o_ref[...] = acc + b_ref[...]


def kernel(latent_rep, output_batch, latent_lib, w, b):
    n, h = latent_rep.shape
    k_pad, d_out = w.shape
    s_width = k_pad - h

    s = jnp.concatenate([output_batch, latent_lib], axis=1)
    pad = s_width - s.shape[1]
    if pad:
        s = jnp.pad(s, ((0, 0), (0, pad)))
    b2 = b.reshape(1, d_out)

    tm = min(1024, n)
    gi = pl.cdiv(n, tm)

    return pl.pallas_call(
        _fused_kernel,
        out_shape=jax.ShapeDtypeStruct((n, d_out), latent_rep.dtype),
        grid=(gi,),
        in_specs=[
            pl.BlockSpec((tm, h), lambda i: (i, 0)),
            pl.BlockSpec((tm, s_width), lambda i: (i, 0)),
            pl.BlockSpec((k_pad, d_out), lambda i: (0, 0)),
            pl.BlockSpec((1, d_out), lambda i: (0, 0)),
        ],
        out_specs=pl.BlockSpec((tm, d_out), lambda i: (i, 0)),
        compiler_params=pltpu.CompilerParams(
            dimension_semantics=("parallel",),
            vmem_limit_bytes=64 << 20,
        ),
    )(latent_rep, s, w, b2)


# speedup vs baseline: 1.2866x; 1.0006x over previous
import jax
import jax.numpy as jnp
from jax import lax
from jax.experimental import pallas as pl
from jax.experimental.pallas import tpu as pltpu


def _fused_kernel(a_ref, s_ref, w_ref, b_ref, o_ref):
    x = jnp.concatenate([a_ref[...], s_ref[...]], axis=1).astype(jnp.bfloat16)
    wv = w_ref[...].astype(jnp.bfloat16)
    acc = lax.dot_general(
        x, wv,
        dimension_numbers=(((1,), (0,)), ((), ())),
        preferred_element_type=jnp.float32)
    o_ref[...] = acc + b_ref[...]


def kernel(latent_rep, output_batch, latent_lib, w, b):
    n, h = latent_rep.shape
    k_pad, d_out = w.shape
    s_width = k_pad - h

    s = jnp.concatenate([output_batch, latent_lib], axis=1)
    pad = s_width - s.shape[1]
    if pad:
        s = jnp.pad(s, ((0, 0), (0, pad)))
    b2 = b.reshape(1, d_out)

    tm = min(1368, n)
    gi = pl.cdiv(n, tm)

    return pl.pallas_call(
        _fused_kernel,
        out_shape=jax.ShapeDtypeStruct((n, d_out), latent_rep.dtype),
        grid=(gi,),
        in_specs=[
            pl.BlockSpec((tm, h), lambda i: (i, 0)),
            pl.BlockSpec((tm, s_width), lambda i: (i, 0)),
            pl.BlockSpec((k_pad, d_out), lambda i: (0, 0)),
            pl.BlockSpec((1, d_out), lambda i: (0, 0)),
        ],
        out_specs=pl.BlockSpec((tm, d_out), lambda i: (i, 0)),
        compiler_params=pltpu.CompilerParams(
            dimension_semantics=("parallel",),
            vmem_limit_bytes=64 << 20,
        ),
    )(latent_rep, s, w, b2)


# speedup vs baseline: 1.2921x; 1.0043x over previous
import jax
import jax.numpy as jnp
from jax import lax
from jax.experimental import pallas as pl
from jax.experimental.pallas import tpu as pltpu


def _fused_kernel(a_ref, s_ref, w_ref, b_ref, o_ref):
    x = jnp.concatenate([a_ref[...], s_ref[...]], axis=1).astype(jnp.bfloat16)
    wv = w_ref[...].astype(jnp.bfloat16)
    acc = lax.dot_general(
        x, wv,
        dimension_numbers=(((1,), (0,)), ((), ())),
        preferred_element_type=jnp.float32)
    o_ref[...] = acc + b_ref[...]


def kernel(latent_rep, output_batch, latent_lib, w, b):
    n, h = latent_rep.shape
    k_pad, d_out = w.shape
    s_width = k_pad - h

    s = jnp.concatenate([output_batch, latent_lib], axis=1)
    pad = s_width - s.shape[1]
    if pad:
        s = jnp.pad(s, ((0, 0), (0, pad)))
    b2 = b.reshape(1, d_out)

    tm = min(1024, n)
    gi = pl.cdiv(n, tm)

    return pl.pallas_call(
        _fused_kernel,
        out_shape=jax.ShapeDtypeStruct((n, d_out), latent_rep.dtype),
        grid=(gi,),
        in_specs=[
            pl.BlockSpec((tm, h), lambda i: (i, 0)),
            pl.BlockSpec((tm, s_width), lambda i: (i, 0)),
            pl.BlockSpec((k_pad, d_out), lambda i: (0, 0)),
            pl.BlockSpec((1, d_out), lambda i: (0, 0)),
        ],
        out_specs=pl.BlockSpec((tm, d_out), lambda i: (i, 0)),
        compiler_params=pltpu.CompilerParams(
            dimension_semantics=("parallel",),
            vmem_limit_bytes=64 << 20,
        ),
    )(latent_rep, s, w, b2)
